# baseline (speedup 1.0000x reference)
import functools
import jax
import jax.numpy as jnp
from jax import lax
from jax.experimental import pallas as pl
from jax.experimental.pallas import tpu as pltpu
from jax.experimental.pallas import tpu_sc as plsc

N_CTX = 20000
TILE = 512
N_PAD = 20480
GRID = N_PAD // TILE
NBLK = N_PAD // 128
K = 100
KPAD = 128
UNIT = 2.0
BIG = 1 << 30


def _heads_decode_kernel(x_ref, w_ref, b_ref, sal_ref, y_ref, score_ref,
                         inds_ref, c_ref, kept_ref):
    i = pl.program_id(0)
    xb = x_ref[...].reshape(8 * TILE, 256)
    y = lax.dot_general(xb, w_ref[...], (((1,), (0,)), ((), ())),
                        preferred_element_type=jnp.float32)
    y_ref[...] = y
    c_logit = y[:, 0:1].reshape(8, TILE) + b_ref[:, 0:1]
    col = i * TILE + lax.broadcasted_iota(jnp.int32, (8, TILE), 1)
    ok = (sal_ref[...] >= 0) & (col < N_CTX)
    c_ref[:, pl.ds(pl.multiple_of(i * TILE, TILE), TILE)] = jnp.where(
        ok, jax.nn.sigmoid(c_logit), 0.0)

    @pl.when(i == GRID - 1)
    def _decode():
        _decode_body(c_ref, score_ref, inds_ref, kept_ref)


def _decode_body(c_ref, score_ref, inds_ref, kept_ref):
    c = c_ref[...]
    colN = lax.broadcasted_iota(jnp.int32, (8, N_PAD), 1)
    r = pltpu.roll(c, shift=N_PAD - 1, axis=1)
    l = pltpu.roll(c, shift=1, axis=1)
    r = jnp.where(colN == N_PAD - 1, 0.0, r)
    l = jnp.where(colN == 0, 0.0, l)
    hmax = jnp.maximum(c, jnp.maximum(l, r))
    kept = jnp.where(hmax == c, c, 0.0)

    kept_ref[...] = kept
    lane = lax.broadcasted_iota(jnp.int32, (8, KPAD), 1)

    def body(i, carry):
        sc, ii = carry
        kept = kept_ref[...]
        m = jnp.max(kept, axis=1, keepdims=True)
        idx = jnp.min(jnp.where(kept == m, colN, BIG), axis=1,
                      keepdims=True)
        kept_ref[...] = jnp.where(colN == idx, -1.0, kept)
        here = lane == i
        sc = jnp.where(here, m, sc)
        ii = jnp.where(here, idx, ii)
        return sc, ii

    zf = jnp.zeros((8, KPAD), jnp.float32)
    zi = jnp.zeros((8, KPAD), jnp.int32)
    sc, ii = lax.fori_loop(0, K, body, (zf, zi))
    score_ref[...] = sc[:, :K]
    inds_ref[...] = ii


def _gather_body(inds_hbm, y_hbm, bias_hbm, left_hbm, right_hbm,
                 idx_v, bias_v, out_v, sem):
    wid = lax.axis_index("s")
    base = wid * 64
    row = base // KPAD
    pltpu.sync_copy(inds_hbm.at[pl.ds(base, 64)], idx_v)
    pltpu.sync_copy(bias_hbm, bias_v)
    for k in range(4):
        n = idx_v[pl.ds(k * 16, 16)]
        tile = lax.shift_right_logical(n, 9)
        rem = jnp.bitwise_and(n, TILE - 1)
        p = tile * (TILE * 8) + row * TILE + rem
        pltpu.async_copy(y_hbm.at[p * 3 + 1], out_v, sem).wait()
        wv = out_v[...] + bias_v[pl.ds(0, 16)]
        pltpu.async_copy(y_hbm.at[p * 3 + 2], out_v, sem).wait()
        ov = out_v[...] + bias_v[pl.ds(16, 16)]
        off = jnp.maximum(ov, 0.0)
        win = jnp.maximum(wv, 0.0)
        center = n.astype(jnp.float32) + off
        left = jnp.clip(center - win / 2.0, 0.0, N_CTX - 1.0) * UNIT
        right = jnp.clip(center + win / 2.0, 0.0, N_CTX - 1.0) * UNIT + UNIT
        out_v[...] = left
        pltpu.sync_copy(out_v, left_hbm.at[pl.ds(base + k * 16, 16)])
        out_v[...] = right
        pltpu.sync_copy(out_v, right_hbm.at[pl.ds(base + k * 16, 16)])


def _sc_gather(inds_flat, y_flat, bias32):
    mesh = plsc.VectorSubcoreMesh(core_axis_name="c", subcore_axis_name="s",
                                  num_cores=1)
    kern = functools.partial(
        pl.kernel, mesh=mesh,
        out_type=[jax.ShapeDtypeStruct((8 * KPAD,), jnp.float32)] * 2,
        scratch_types=[
            pltpu.VMEM((64,), jnp.int32),
            pltpu.VMEM((32,), jnp.float32),
            pltpu.VMEM((16,), jnp.float32),
            pltpu.SemaphoreType.DMA,
        ],
    )(_gather_body)
    return kern(inds_flat, y_flat, bias32)


@jax.jit
def kernel(x, saliency, Wc, bc, Ww, bw, Wo, bo):
    w = jnp.concatenate([Wc, Ww, Wo], axis=1)
    b = jnp.broadcast_to(jnp.stack([bc[0], bw[0], bo[0]])[None, :], (8, 3))
    y_all, score, inds = pl.pallas_call(
        _heads_decode_kernel,
        grid=(GRID,),
        in_specs=[
            pl.BlockSpec((8, TILE, 256), lambda i: (0, i, 0)),
            pl.BlockSpec((256, 3), lambda i: (0, 0)),
            pl.BlockSpec((8, 3), lambda i: (0, 0)),
            pl.BlockSpec((8, TILE), lambda i: (0, i)),
        ],
        out_specs=[pl.BlockSpec((8 * TILE, 3), lambda i: (i, 0)),
                   pl.BlockSpec((8, K), lambda i: (0, 0)),
                   pl.BlockSpec((8, KPAD), lambda i: (0, 0))],
        out_shape=[jax.ShapeDtypeStruct((GRID * 8 * TILE, 3), jnp.float32),
                   jax.ShapeDtypeStruct((8, K), jnp.float32),
                   jax.ShapeDtypeStruct((8, KPAD), jnp.int32)],
        scratch_shapes=[pltpu.VMEM((8, N_PAD), jnp.float32),
                        pltpu.VMEM((8, N_PAD), jnp.float32)],
    )(x, w, b, saliency)
    bias32 = jnp.concatenate([jnp.broadcast_to(bw, (16,)),
                              jnp.broadcast_to(bo, (16,))])
    left, right = _sc_gather(inds.reshape(8 * KPAD), y_all.reshape(-1),
                             bias32)
    left = left.reshape(8, KPAD)[:, :K]
    right = right.reshape(8, KPAD)[:, :K]
    return jnp.stack([left, right, score[:, :K]], axis=2)


# speedup vs baseline: 1.0173x; 1.0173x over previous
import functools
import jax
import jax.numpy as jnp
from jax import lax
from jax.experimental import pallas as pl
from jax.experimental.pallas import tpu as pltpu
from jax.experimental.pallas import tpu_sc as plsc

N_CTX = 20000
TILE = 512
N_PAD = 20480
GRID = N_PAD // TILE
NBLK = N_PAD // 128
K = 100
KPAD = 128
UNIT = 2.0
BIG = 1 << 30


def _heads_decode_kernel(x_ref, w_ref, b_ref, sal_ref, y_ref, score_ref,
                         inds_ref, c_ref, kept_ref):
    i = pl.program_id(0)
    xb = x_ref[...].reshape(8 * TILE, 256)
    y = lax.dot_general(xb, w_ref[...], (((1,), (0,)), ((), ())),
                        preferred_element_type=jnp.float32)
    y_ref[...] = y
    c_logit = y[:, 0:1].reshape(8, TILE) + b_ref[:, 0:1]
    col = i * TILE + lax.broadcasted_iota(jnp.int32, (8, TILE), 1)
    ok = (sal_ref[...] >= 0) & (col < N_CTX)
    c_ref[:, pl.ds(pl.multiple_of(i * TILE, TILE), TILE)] = jnp.where(
        ok, jax.nn.sigmoid(c_logit), 0.0)

    @pl.when(i == GRID - 1)
    def _decode():
        _decode_body(c_ref, score_ref, inds_ref, kept_ref)


def _decode_body(c_ref, score_ref, inds_ref, kept_ref):
    c = c_ref[...]
    colN = lax.broadcasted_iota(jnp.int32, (8, N_PAD), 1)
    r = pltpu.roll(c, shift=N_PAD - 1, axis=1)
    l = pltpu.roll(c, shift=1, axis=1)
    r = jnp.where(colN == N_PAD - 1, 0.0, r)
    l = jnp.where(colN == 0, 0.0, l)
    hmax = jnp.maximum(c, jnp.maximum(l, r))
    kept = jnp.where(hmax == c, c, 0.0)

    kept_ref[...] = kept
    lane = lax.broadcasted_iota(jnp.int32, (8, KPAD), 1)

    def body(i, carry):
        sc, ii = carry
        kept = kept_ref[...]
        m1 = jnp.max(kept, axis=1, keepdims=True)
        i1 = jnp.min(jnp.where(kept == m1, colN, BIG), axis=1,
                     keepdims=True)
        k2 = jnp.where(colN == i1, -1.0, kept)
        m2 = jnp.max(k2, axis=1, keepdims=True)
        i2 = jnp.min(jnp.where(k2 == m2, colN, BIG), axis=1,
                     keepdims=True)
        kept_ref[...] = jnp.where(colN == i2, -1.0, k2)
        h1 = lane == 2 * i
        h2 = lane == 2 * i + 1
        sc = jnp.where(h1, m1, jnp.where(h2, m2, sc))
        ii = jnp.where(h1, i1, jnp.where(h2, i2, ii))
        return sc, ii

    zf = jnp.zeros((8, KPAD), jnp.float32)
    zi = jnp.zeros((8, KPAD), jnp.int32)
    sc, ii = lax.fori_loop(0, K // 2, body, (zf, zi))
    score_ref[...] = sc[:, :K]
    inds_ref[...] = ii


def _gather_body(inds_hbm, y_hbm, bias_hbm, left_hbm, right_hbm,
                 idx_v, bias_v, out_v, sem):
    wid = lax.axis_index("s")
    base = wid * 64
    row = base // KPAD
    pltpu.sync_copy(inds_hbm.at[pl.ds(base, 64)], idx_v)
    pltpu.sync_copy(bias_hbm, bias_v)
    for k in range(4):
        n = idx_v[pl.ds(k * 16, 16)]
        tile = lax.shift_right_logical(n, 9)
        rem = jnp.bitwise_and(n, TILE - 1)
        p = tile * (TILE * 8) + row * TILE + rem
        pltpu.async_copy(y_hbm.at[p * 3 + 1], out_v, sem).wait()
        wv = out_v[...] + bias_v[pl.ds(0, 16)]
        pltpu.async_copy(y_hbm.at[p * 3 + 2], out_v, sem).wait()
        ov = out_v[...] + bias_v[pl.ds(16, 16)]
        off = jnp.maximum(ov, 0.0)
        win = jnp.maximum(wv, 0.0)
        center = n.astype(jnp.float32) + off
        left = jnp.clip(center - win / 2.0, 0.0, N_CTX - 1.0) * UNIT
        right = jnp.clip(center + win / 2.0, 0.0, N_CTX - 1.0) * UNIT + UNIT
        out_v[...] = left
        pltpu.sync_copy(out_v, left_hbm.at[pl.ds(base + k * 16, 16)])
        out_v[...] = right
        pltpu.sync_copy(out_v, right_hbm.at[pl.ds(base + k * 16, 16)])


def _sc_gather(inds_flat, y_flat, bias32):
    mesh = plsc.VectorSubcoreMesh(core_axis_name="c", subcore_axis_name="s",
                                  num_cores=1)
    kern = functools.partial(
        pl.kernel, mesh=mesh,
        out_type=[jax.ShapeDtypeStruct((8 * KPAD,), jnp.float32)] * 2,
        scratch_types=[
            pltpu.VMEM((64,), jnp.int32),
            pltpu.VMEM((32,), jnp.float32),
            pltpu.VMEM((16,), jnp.float32),
            pltpu.SemaphoreType.DMA,
        ],
    )(_gather_body)
    return kern(inds_flat, y_flat, bias32)


@jax.jit
def kernel(x, saliency, Wc, bc, Ww, bw, Wo, bo):
    w = jnp.concatenate([Wc, Ww, Wo], axis=1)
    b = jnp.broadcast_to(jnp.stack([bc[0], bw[0], bo[0]])[None, :], (8, 3))
    y_all, score, inds = pl.pallas_call(
        _heads_decode_kernel,
        grid=(GRID,),
        in_specs=[
            pl.BlockSpec((8, TILE, 256), lambda i: (0, i, 0)),
            pl.BlockSpec((256, 3), lambda i: (0, 0)),
            pl.BlockSpec((8, 3), lambda i: (0, 0)),
            pl.BlockSpec((8, TILE), lambda i: (0, i)),
        ],
        out_specs=[pl.BlockSpec((8 * TILE, 3), lambda i: (i, 0)),
                   pl.BlockSpec((8, K), lambda i: (0, 0)),
                   pl.BlockSpec((8, KPAD), lambda i: (0, 0))],
        out_shape=[jax.ShapeDtypeStruct((GRID * 8 * TILE, 3), jnp.float32),
                   jax.ShapeDtypeStruct((8, K), jnp.float32),
                   jax.ShapeDtypeStruct((8, KPAD), jnp.int32)],
        scratch_shapes=[pltpu.VMEM((8, N_PAD), jnp.float32),
                        pltpu.VMEM((8, N_PAD), jnp.float32)],
    )(x, w, b, saliency)
    bias32 = jnp.concatenate([jnp.broadcast_to(bw, (16,)),
                              jnp.broadcast_to(bo, (16,))])
    left, right = _sc_gather(inds.reshape(8 * KPAD), y_all.reshape(-1),
                             bias32)
    left = left.reshape(8, KPAD)[:, :K]
    right = right.reshape(8, KPAD)[:, :K]
    return jnp.stack([left, right, score[:, :K]], axis=2)
